# Initial kernel scaffold; baseline (speedup 1.0000x reference)
import jax
import jax.numpy as jnp
from jax.experimental import pallas as pl


def kernel(x, row_idx, u_table, W, b):
    raise NotImplementedError("write your pallas kernel here")


# speedup vs baseline: 1.5159x; 1.5159x over previous
import functools

import jax
import jax.numpy as jnp
from jax import lax
from jax.experimental import pallas as pl
from jax.experimental.pallas import tpu as pltpu
from jax.experimental.pallas import tpu_sc as plsc

B = 16384
DX = 120
DU = 8
NV = 600

_NC, _NS = 2, 16
_NW = _NC * _NS
_BPW = B // _NW


def _sc_gather(idx_hbm, table_hbm, u_hbm, idx_v, rows_v, sem):
    wid = lax.axis_index("s") * _NC + lax.axis_index("c")
    base = wid * _BPW
    pltpu.sync_copy(idx_hbm.at[pl.ds(base, _BPW)], idx_v)
    pltpu.async_copy(table_hbm.at[idx_v], rows_v, sem).wait()
    pltpu.sync_copy(rows_v, u_hbm.at[pl.ds(base, _BPW)])


def _make_sc_gather_call():
    return functools.partial(
        pl.kernel,
        mesh=plsc.VectorSubcoreMesh(core_axis_name="c", subcore_axis_name="s"),
        out_type=jax.ShapeDtypeStruct((B, DU), jnp.float32),
        scratch_types=[
            pltpu.VMEM((_BPW,), jnp.int32),
            pltpu.VMEM((_BPW, DU), jnp.float32),
            pltpu.SemaphoreType.DMA,
        ],
        compiler_params=pltpu.CompilerParams(use_tc_tiling_on_sc=False),
    )(_sc_gather)


def _tc_matvec(x_ref, u_ref, w_ref, b_ref, out_ref):
    wavg = jnp.sum(w_ref[...], axis=0, keepdims=True) * 0.25
    b_avg = jnp.sum(b_ref[...]) * 0.25
    sx = jnp.sum(x_ref[...] * wavg[:, :DX], axis=1, keepdims=True)
    su = jnp.sum(u_ref[...] * wavg[:, DX:], axis=1, keepdims=True)
    out_ref[...] = sx + su + b_avg


def kernel(x, row_idx, u_table, W, b):
    idx = row_idx.astype(jnp.int32)
    u = _make_sc_gather_call()(idx, u_table)

    blk = 2048
    grid = B // blk
    out2d = pl.pallas_call(
        _tc_matvec,
        grid=(grid,),
        in_specs=[
            pl.BlockSpec((blk, DX), lambda i: (i, 0)),
            pl.BlockSpec((blk, DU), lambda i: (i, 0)),
            pl.BlockSpec((4, 128), lambda i: (0, 0)),
            pl.BlockSpec((4,), lambda i: (0,)),
        ],
        out_specs=pl.BlockSpec((blk, 1), lambda i: (i, 0)),
        out_shape=jax.ShapeDtypeStruct((B, 1), jnp.float32),
    )(x, u, W, b)
    return (out2d.reshape(-1), u)


# speedup vs baseline: 2.5360x; 1.6729x over previous
import functools

import jax
import jax.numpy as jnp
from jax import lax
from jax.experimental import pallas as pl
from jax.experimental.pallas import tpu as pltpu
from jax.experimental.pallas import tpu_sc as plsc

B = 16384
DX = 120
DU = 8
NV = 600

_NC, _NS = 2, 16
_NW = _NC * _NS
_BPW = B // _NW


def _sc_fused(idx_hbm, table_hbm, w_hbm, ut_hbm, su_hbm, idx_v, rows_v, w_v,
              wtail_v, rowst_v, su_v, sem):
    wid = lax.axis_index("s") * _NC + lax.axis_index("c")
    base = wid * _BPW
    pltpu.sync_copy(idx_hbm.at[pl.ds(base, _BPW)], idx_v)
    pltpu.sync_copy(w_hbm, w_v)
    pltpu.async_copy(table_hbm.at[idx_v], rows_v, sem).wait()

    vsum = (w_v[0, pl.ds(112, 16)] + w_v[1, pl.ds(112, 16)]
            + w_v[2, pl.ds(112, 16)] + w_v[3, pl.ds(112, 16)]) * 0.25
    wtail_v[...] = vsum
    wcs = [
        plsc.load_gather(wtail_v, [jnp.full((16,), 8 + c, jnp.int32)])
        for c in range(DU)
    ]
    lanes = lax.iota(jnp.int32, 16)

    def body(g, carry):
        row0 = g * 16
        acc = jnp.zeros((16,), jnp.float32)
        for c in range(DU):
            vals = plsc.load_gather(
                rows_v, [row0 + lanes, jnp.full((16,), c, jnp.int32)])
            acc = acc + vals * wcs[c]
            rowst_v[c, pl.ds(row0, 16)] = vals
        su_v[pl.ds(row0, 16)] = acc
        return carry

    lax.fori_loop(0, _BPW // 16, body, 0)
    pltpu.sync_copy(su_v, su_hbm.at[pl.ds(base, _BPW)])
    for c in range(DU):
        pltpu.sync_copy(rowst_v.at[c], ut_hbm.at[c, pl.ds(base, _BPW)])


def _make_sc_call():
    return functools.partial(
        pl.kernel,
        mesh=plsc.VectorSubcoreMesh(core_axis_name="c", subcore_axis_name="s"),
        out_type=(
            jax.ShapeDtypeStruct((DU, B), jnp.float32),
            jax.ShapeDtypeStruct((B,), jnp.float32),
        ),
        scratch_types=[
            pltpu.VMEM((_BPW,), jnp.int32),
            pltpu.VMEM((_BPW, DU), jnp.float32),
            pltpu.VMEM((4, 128), jnp.float32),
            pltpu.VMEM((16,), jnp.float32),
            pltpu.VMEM((DU, _BPW), jnp.float32),
            pltpu.VMEM((_BPW,), jnp.float32),
            pltpu.SemaphoreType.DMA,
        ],
        compiler_params=pltpu.CompilerParams(
            use_tc_tiling_on_sc=False, needs_layout_passes=False),
    )(_sc_fused)


def _tc_matvec(xt_ref, w_ref, b_ref, su_ref, out_ref):
    wavg = jnp.sum(w_ref[...], axis=0, keepdims=True) * 0.25
    b_avg = jnp.sum(b_ref[...]) * 0.25
    sx = lax.dot_general(
        wavg[:, :DX], xt_ref[...], (((1,), (0,)), ((), ())),
        preferred_element_type=jnp.float32)
    out_ref[...] = sx[0] + su_ref[...] + b_avg


def kernel(x, row_idx, u_table, W, b):
    idx = row_idx.astype(jnp.int32)
    ut, su = _make_sc_call()(idx, u_table, W)
    u = ut.T

    xt = x.T
    blk = 2048
    out = pl.pallas_call(
        _tc_matvec,
        grid=(B // blk,),
        in_specs=[
            pl.BlockSpec((DX, blk), lambda i: (0, i)),
            pl.BlockSpec((4, 128), lambda i: (0, 0)),
            pl.BlockSpec((4,), lambda i: (0,)),
            pl.BlockSpec((blk,), lambda i: (i,)),
        ],
        out_specs=pl.BlockSpec((blk,), lambda i: (i,)),
        out_shape=jax.ShapeDtypeStruct((B,), jnp.float32),
    )(xt, W, b, su)
    return (out, u)


# speedup vs baseline: 2.9493x; 1.1630x over previous
import functools

import jax
import jax.numpy as jnp
from jax import lax
from jax.experimental import pallas as pl
from jax.experimental.pallas import tpu as pltpu
from jax.experimental.pallas import tpu_sc as plsc

B = 16384
DX = 120
DU = 8
NV = 600

_NC, _NS = 2, 16
_NW = _NC * _NS
_BPW = B // _NW


def _sc_fused(idx_hbm, tabt_hbm, w_hbm, ut_hbm, su_hbm, idx_v, tab_v, w_v,
              wtail_v, rowst_v, su_v):
    wid = lax.axis_index("s") * _NC + lax.axis_index("c")
    base = wid * _BPW
    pltpu.sync_copy(idx_hbm.at[pl.ds(base, _BPW)], idx_v)
    pltpu.sync_copy(tabt_hbm, tab_v)
    pltpu.sync_copy(w_hbm, w_v)

    vsum = (w_v[0, pl.ds(112, 16)] + w_v[1, pl.ds(112, 16)]
            + w_v[2, pl.ds(112, 16)] + w_v[3, pl.ds(112, 16)]) * 0.25
    wtail_v[...] = vsum
    wcs = [
        plsc.load_gather(wtail_v, [jnp.full((16,), 8 + c, jnp.int32)])
        for c in range(DU)
    ]

    def body(g, carry):
        row0 = g * 16
        idx16 = idx_v[pl.ds(row0, 16)]
        acc = jnp.zeros((16,), jnp.float32)
        for c in range(DU):
            vals = plsc.load_gather(
                tab_v, [jnp.full((16,), c, jnp.int32), idx16])
            acc = acc + vals * wcs[c]
            rowst_v[c, pl.ds(row0, 16)] = vals
        su_v[pl.ds(row0, 16)] = acc
        return carry

    lax.fori_loop(0, _BPW // 16, body, 0)
    pltpu.sync_copy(su_v, su_hbm.at[pl.ds(base, _BPW)])
    for c in range(DU):
        pltpu.sync_copy(rowst_v.at[c], ut_hbm.at[c, pl.ds(base, _BPW)])


def _make_sc_call():
    return functools.partial(
        pl.kernel,
        mesh=plsc.VectorSubcoreMesh(core_axis_name="c", subcore_axis_name="s"),
        out_type=(
            jax.ShapeDtypeStruct((DU, B), jnp.float32),
            jax.ShapeDtypeStruct((B,), jnp.float32),
        ),
        scratch_types=[
            pltpu.VMEM((_BPW,), jnp.int32),
            pltpu.VMEM((DU, NV), jnp.float32),
            pltpu.VMEM((4, 128), jnp.float32),
            pltpu.VMEM((16,), jnp.float32),
            pltpu.VMEM((DU, _BPW), jnp.float32),
            pltpu.VMEM((_BPW,), jnp.float32),
        ],
        compiler_params=pltpu.CompilerParams(
            use_tc_tiling_on_sc=False, needs_layout_passes=False),
    )(_sc_fused)


def _tc_matvec(xt_ref, w_ref, b_ref, out_ref):
    wavg = jnp.sum(w_ref[...], axis=0, keepdims=True) * 0.25
    b_avg = jnp.sum(b_ref[...]) * 0.25
    sx = lax.dot_general(
        wavg[:, :DX], xt_ref[...], (((1,), (0,)), ((), ())),
        preferred_element_type=jnp.float32)
    out_ref[...] = sx[0] + b_avg


def _tc_add(a_ref, b_ref, out_ref):
    out_ref[...] = a_ref[...] + b_ref[...]


def kernel(x, row_idx, u_table, W, b):
    idx = row_idx.astype(jnp.int32)
    ut, su = _make_sc_call()(idx, u_table.T, W)
    u = ut.T

    xt = x.T
    blk = 2048
    partial = pl.pallas_call(
        _tc_matvec,
        grid=(B // blk,),
        in_specs=[
            pl.BlockSpec((DX, blk), lambda i: (0, i)),
            pl.BlockSpec((4, 128), lambda i: (0, 0)),
            pl.BlockSpec((4,), lambda i: (0,)),
        ],
        out_specs=pl.BlockSpec((blk,), lambda i: (i,)),
        out_shape=jax.ShapeDtypeStruct((B,), jnp.float32),
    )(xt, W, b)

    out = pl.pallas_call(
        _tc_add,
        in_specs=[
            pl.BlockSpec((B,), lambda: (0,)),
            pl.BlockSpec((B,), lambda: (0,)),
        ],
        out_specs=pl.BlockSpec((B,), lambda: (0,)),
        out_shape=jax.ShapeDtypeStruct((B,), jnp.float32),
    )(partial, su)
    return (out, u)


# speedup vs baseline: 3.0676x; 1.0401x over previous
import functools

import jax
import jax.numpy as jnp
from jax import lax
from jax.experimental import pallas as pl
from jax.experimental.pallas import tpu as pltpu
from jax.experimental.pallas import tpu_sc as plsc

B = 16384
DX = 120
DU = 8
NV = 600

_NC, _NS = 2, 16
_NW = _NC * _NS
_BPW = B // _NW


def _sc_fused(idx_hbm, tabt_hbm, w_hbm, ut_hbm, su_hbm, idx_v, tab_v, w_v,
              wtail_v, rowst_v, su_v):
    wid = lax.axis_index("s") * _NC + lax.axis_index("c")
    base = wid * _BPW
    pltpu.sync_copy(idx_hbm.at[pl.ds(base, _BPW)], idx_v)
    pltpu.sync_copy(tabt_hbm, tab_v)
    pltpu.sync_copy(w_hbm, w_v)

    vsum = (w_v[0, pl.ds(112, 16)] + w_v[1, pl.ds(112, 16)]
            + w_v[2, pl.ds(112, 16)] + w_v[3, pl.ds(112, 16)]) * 0.25
    wtail_v[...] = vsum
    wcs = [
        plsc.load_gather(wtail_v, [jnp.full((16,), 8 + c, jnp.int32)])
        for c in range(DU)
    ]

    cvecs = [jnp.full((16,), c, jnp.int32) for c in range(DU)]

    @plsc.parallel_loop(0, _BPW, step=16, unroll=4)
    def _loop(row0):
        idx16 = idx_v[pl.ds(row0, 16)]
        acc = jnp.zeros((16,), jnp.float32)
        for c in range(DU):
            vals = plsc.load_gather(tab_v, [cvecs[c], idx16])
            acc = acc + vals * wcs[c]
            rowst_v[c, pl.ds(row0, 16)] = vals
        su_v[pl.ds(row0, 16)] = acc
    pltpu.sync_copy(su_v, su_hbm.at[pl.ds(base, _BPW)])
    for c in range(DU):
        pltpu.sync_copy(rowst_v.at[c], ut_hbm.at[c, pl.ds(base, _BPW)])


def _make_sc_call():
    return functools.partial(
        pl.kernel,
        mesh=plsc.VectorSubcoreMesh(core_axis_name="c", subcore_axis_name="s"),
        out_type=(
            jax.ShapeDtypeStruct((DU, B), jnp.float32),
            jax.ShapeDtypeStruct((B,), jnp.float32),
        ),
        scratch_types=[
            pltpu.VMEM((_BPW,), jnp.int32),
            pltpu.VMEM((DU, NV), jnp.float32),
            pltpu.VMEM((4, 128), jnp.float32),
            pltpu.VMEM((16,), jnp.float32),
            pltpu.VMEM((DU, _BPW), jnp.float32),
            pltpu.VMEM((_BPW,), jnp.float32),
        ],
        compiler_params=pltpu.CompilerParams(
            use_tc_tiling_on_sc=False, needs_layout_passes=False),
    )(_sc_fused)


def _tc_matvec(xt_ref, w_ref, b_ref, out_ref):
    wavg = jnp.sum(w_ref[...], axis=0, keepdims=True) * 0.25
    b_avg = jnp.sum(b_ref[...]) * 0.25
    sx = lax.dot_general(
        wavg[:, :DX], xt_ref[...], (((1,), (0,)), ((), ())),
        preferred_element_type=jnp.float32)
    out_ref[...] = sx[0] + b_avg


def _tc_add(a_ref, b_ref, out_ref):
    out_ref[...] = a_ref[...] + b_ref[...]


def kernel(x, row_idx, u_table, W, b):
    idx = row_idx.astype(jnp.int32)
    ut, su = _make_sc_call()(idx, u_table.T, W)
    u = ut.T

    xt = x.T
    blk = 4096
    partial = pl.pallas_call(
        _tc_matvec,
        grid=(B // blk,),
        in_specs=[
            pl.BlockSpec((DX, blk), lambda i: (0, i)),
            pl.BlockSpec((4, 128), lambda i: (0, 0)),
            pl.BlockSpec((4,), lambda i: (0,)),
        ],
        out_specs=pl.BlockSpec((blk,), lambda i: (i,)),
        out_shape=jax.ShapeDtypeStruct((B,), jnp.float32),
    )(xt, W, b)

    out = pl.pallas_call(
        _tc_add,
        in_specs=[
            pl.BlockSpec((B,), lambda: (0,)),
            pl.BlockSpec((B,), lambda: (0,)),
        ],
        out_specs=pl.BlockSpec((B,), lambda: (0,)),
        out_shape=jax.ShapeDtypeStruct((B,), jnp.float32),
    )(partial, su)
    return (out, u)
